# baseline (speedup 1.0000x reference)
import functools

import jax
import jax.numpy as jnp
from jax import lax
from jax.experimental import pallas as pl
from jax.experimental.pallas import tpu as pltpu
from jax.experimental.pallas import tpu_sc as plsc

NCORES = 2
NSUB = 16
LANES = 16
CHUNK = 80
BM = 1000
NUM_LAYERS = 3


def _elu(v):
    return jnp.where(v > 0, v, jnp.exp(v) - 1.0)


def _alphas_and_gmax(i, h, a_src_ref, a_dst_ref, as_ref, ad_ref, g_ref):
    as_blk = jnp.dot(h, a_src_ref[0, :], preferred_element_type=jnp.float32)
    ad_blk = jnp.dot(h, a_dst_ref[0, :], preferred_element_type=jnp.float32)
    as_ref[0, 0, :] = as_blk
    ad_ref[0, 0, :] = ad_blk

    @pl.when(i == 0)
    def _():
        g_ref[...] = jnp.full((8, 128), -jnp.inf, jnp.float32)

    g_ref[...] = jnp.maximum(g_ref[...], jnp.full((8, 128), jnp.max(as_blk)))


def _prep_x_body(x_ref, w_ref, a_src_ref, a_dst_ref, h_ref, as_ref, ad_ref, g_ref):
    i = pl.program_id(0)
    h = jnp.dot(x_ref[...], w_ref[...], preferred_element_type=jnp.float32)
    h_ref[...] = h
    _alphas_and_gmax(i, h, a_src_ref, a_dst_ref, as_ref, ad_ref, g_ref)


def _prep_p_body(p_ref0, p_ref1, d_ref0, d_ref1, w_ref, a_src_ref, a_dst_ref,
                 h_ref, as_ref, ad_ref, g_ref):
    i = pl.program_id(0)
    num = p_ref0[0, :, :] + p_ref1[0, :, :]
    den = d_ref0[0, :, :] + d_ref1[0, :, :] + 1e-16
    x = _elu(num / den)
    h = jnp.dot(x, w_ref[...], preferred_element_type=jnp.float32)
    h_ref[...] = h
    _alphas_and_gmax(i, h, a_src_ref, a_dst_ref, as_ref, ad_ref, g_ref)


def _combine_body(p_ref0, p_ref1, d_ref0, d_ref1, o_ref):
    num = p_ref0[0, :, :] + p_ref1[0, :, :]
    den = d_ref0[0, :, :] + d_ref1[0, :, :] + 1e-16
    o_ref[...] = _elu(num / den)


def _make_tc_kernels(n, d, np_pad):
    nb = n // BM
    w_spec = pl.BlockSpec((d, d), lambda i: (0, 0))
    a_spec = pl.BlockSpec((1, d), lambda i: (0, 0))
    x_spec = pl.BlockSpec((BM, d), lambda i: (i, 0))
    p0_spec = pl.BlockSpec((1, BM, d), lambda i: (0, i, 0))
    p1_spec = pl.BlockSpec((1, BM, d), lambda i: (1, i, 0))
    d0_spec = pl.BlockSpec((1, BM, 1), lambda i: (0, i, 0))
    d1_spec = pl.BlockSpec((1, BM, 1), lambda i: (1, i, 0))
    al_spec = pl.BlockSpec((1, 1, BM), lambda i: (i, 0, 0))
    g_spec = pl.BlockSpec((8, 128), lambda i: (0, 0))

    out_types = [
        jax.ShapeDtypeStruct((n, d), jnp.float32),
        jax.ShapeDtypeStruct((nb, 1, BM), jnp.float32),
        jax.ShapeDtypeStruct((nb, 1, BM), jnp.float32),
        jax.ShapeDtypeStruct((8, 128), jnp.float32),
    ]
    out_specs = [x_spec, al_spec, al_spec, g_spec]

    prep_x = pl.pallas_call(
        _prep_x_body,
        grid=(nb,),
        in_specs=[x_spec, w_spec, a_spec, a_spec],
        out_specs=out_specs,
        out_shape=out_types,
    )
    prep_p = pl.pallas_call(
        _prep_p_body,
        grid=(nb,),
        in_specs=[p0_spec, p1_spec, d0_spec, d1_spec, w_spec, a_spec, a_spec],
        out_specs=out_specs,
        out_shape=out_types,
    )
    combine = pl.pallas_call(
        _combine_body,
        grid=(nb,),
        in_specs=[p0_spec, p1_spec, d0_spec, d1_spec],
        out_specs=x_spec,
        out_shape=jax.ShapeDtypeStruct((n, d), jnp.float32),
    )
    return prep_x, prep_p, combine


GRP = 25


def _make_sc_edge(n, d, e, np_pad):
    per_tile = e // (NCORES * NSUB)
    ngrp = per_tile // (GRP * CHUNK)
    assert ngrp * GRP * CHUNK * NCORES * NSUB == e
    rows_per_tile = np_pad // NSUB
    nzc = rows_per_tile // CHUNK
    assert nzc * CHUNK == rows_per_tile

    mesh = plsc.VectorSubcoreMesh(
        core_axis_name="c", subcore_axis_name="s",
        num_cores=NCORES, num_subcores=NSUB)

    @functools.partial(
        pl.kernel,
        out_type=[
            jax.ShapeDtypeStruct((NCORES, np_pad, d), jnp.float32),
            jax.ShapeDtypeStruct((NCORES, np_pad), jnp.float32),
        ],
        mesh=mesh,
        compiler_params=pltpu.CompilerParams(needs_layout_passes=False),
        scratch_types=[
            pltpu.VMEM((1, 128), jnp.float32),
            pltpu.VMEM((2, GRP, CHUNK), jnp.int32),
            pltpu.VMEM((2, GRP, CHUNK), jnp.int32),
            pltpu.VMEM((2, CHUNK, d), jnp.float32),
            pltpu.VMEM((2, CHUNK), jnp.float32),
            pltpu.VMEM((2, CHUNK), jnp.float32),
            pltpu.VMEM((2, CHUNK), jnp.float32),
            pltpu.VMEM_SHARED((np_pad, d), jnp.float32),
            pltpu.VMEM_SHARED((np_pad,), jnp.float32),
            pltpu.SemaphoreType.DMA((2,)),
            pltpu.SemaphoreType.DMA((2,)),
        ],
    )
    def sc_edge(h_hbm, as_hbm, ad_hbm, g_hbm, src_hbm, dst_hbm,
                p_out, den_out, g_v, src_g, dst_g, rows2, asc2, adc2, ee2,
                p_sp, d_sp, isem, gsem):
        cid = lax.axis_index("c")
        sid = lax.axis_index("s")
        row0 = sid * rows_per_tile
        nch = ngrp * GRP

        def _zrows(i, _):
            for u in range(d // LANES):
                rows2[0, i, pl.ds(u * LANES, LANES)] = jnp.zeros((LANES,), jnp.float32)
            return 0
        lax.fori_loop(0, CHUNK, _zrows, 0)
        for u in range(CHUNK // LANES):
            ee2[0, pl.ds(u * LANES, LANES)] = jnp.zeros((LANES,), jnp.float32)
        for b in range(nzc):
            pltpu.sync_copy(rows2.at[0], p_sp.at[pl.ds(row0 + b * CHUNK, CHUNK)])
            pltpu.sync_copy(ee2.at[0], d_sp.at[pl.ds(row0 + b * CHUNK, CHUNK)])

        pltpu.sync_copy(g_hbm, g_v)
        plsc.subcore_barrier()

        g16 = g_v[0, pl.ds(0, LANES)]

        def issue_idx(gi, parity):
            pltpu.async_copy(src_hbm.at[cid, sid, gi], src_g.at[parity],
                             isem.at[parity])
            pltpu.async_copy(dst_hbm.at[cid, sid, gi], dst_g.at[parity],
                             isem.at[parity])

        def wait_idx(parity):
            pltpu.make_async_copy(src_hbm.at[cid, sid, 0], src_g.at[parity],
                                  isem.at[parity]).wait()
            pltpu.make_async_copy(dst_hbm.at[cid, sid, 0], dst_g.at[parity],
                                  isem.at[parity]).wait()

        def wait_gathers(buf):
            pltpu.make_async_copy(h_hbm.at[src_g.at[0, 0]], rows2.at[buf],
                                  gsem.at[buf]).wait()
            pltpu.make_async_copy(as_hbm.at[src_g.at[0, 0]], asc2.at[buf],
                                  gsem.at[buf]).wait()
            pltpu.make_async_copy(ad_hbm.at[dst_g.at[0, 0]], adc2.at[buf],
                                  gsem.at[buf]).wait()

        def issue_gathers(jc, buf):
            gi = jc // GRP
            jj = jc - gi * GRP
            gp = lax.rem(gi, 2)
            pltpu.async_copy(h_hbm.at[src_g.at[gp, jj]], rows2.at[buf],
                             gsem.at[buf])
            pltpu.async_copy(as_hbm.at[src_g.at[gp, jj]], asc2.at[buf],
                             gsem.at[buf])
            pltpu.async_copy(ad_hbm.at[dst_g.at[gp, jj]], adc2.at[buf],
                             gsem.at[buf])

        issue_idx(0, 0)
        wait_idx(0)
        issue_gathers(0, 0)

        def body(j, _):
            b = lax.rem(j, 2)
            nb = 1 - b
            gi = j // GRP
            jj = j - gi * GRP

            @pl.when(jnp.logical_and(jj == 0, gi + 1 < ngrp))
            def _():
                issue_idx(gi + 1, lax.rem(gi + 1, 2))

            wait_gathers(b)

            @pl.when(j + 1 < nch)
            def _():
                gi1 = (j + 1) // GRP
                jj1 = (j + 1) - gi1 * GRP

                @pl.when(jnp.logical_and(jj1 == 0, gi1 > 0))
                def _():
                    wait_idx(lax.rem(gi1, 2))
                issue_gathers(j + 1, nb)

            for q in range(CHUNK // LANES):
                sl = pl.ds(q * LANES, LANES)
                a_s = asc2[b, sl]
                a_d = adc2[b, sl]
                s = a_s + a_d
                ee = jnp.exp(jnp.where(s > 0, s, 0.2 * s)
                             - jnp.where(a_d + g16 > 0, a_d + g16,
                                         0.2 * (a_d + g16)))
                ee2[b, sl] = ee

            def scale(q, _):
                ee16 = ee2[b, pl.ds(q * LANES, LANES)]
                base = q * LANES
                for r in range(LANES):
                    cf = jnp.full((LANES,), ee16[r], jnp.float32)
                    for u in range(d // LANES):
                        sl = pl.ds(u * LANES, LANES)
                        rows2[b, base + r, sl] = rows2[b, base + r, sl] * cf
                return 0
            lax.fori_loop(0, CHUNK // LANES, scale, 0)

            gp = lax.rem(gi, 2)
            pltpu.sync_copy(rows2.at[b], p_sp.at[dst_g.at[gp, jj]], add=True)
            pltpu.sync_copy(ee2.at[b], d_sp.at[dst_g.at[gp, jj]], add=True)
            return 0
        lax.fori_loop(0, nch, body, 0)
        plsc.subcore_barrier()

        pltpu.sync_copy(p_sp.at[pl.ds(row0, rows_per_tile)],
                        p_out.at[cid, pl.ds(row0, rows_per_tile)])
        pltpu.sync_copy(d_sp.at[pl.ds(row0, rows_per_tile)],
                        den_out.at[cid, pl.ds(row0, rows_per_tile)])

    return sc_edge


def kernel(features, edge_index, W, a_src, a_dst):
    n, d = features.shape
    e = edge_index.shape[1]
    np_pad = ((n + NSUB * CHUNK - 1) // (NSUB * CHUNK)) * (NSUB * CHUNK)

    prep_x, prep_p, combine = _make_tc_kernels(n, d, np_pad)
    sc_edge = _make_sc_edge(n, d, e, np_pad)

    ngrp = e // (NCORES * NSUB * GRP * CHUNK)
    src4 = edge_index[0].reshape(NCORES, NSUB, ngrp, GRP, CHUNK)
    dst4 = edge_index[1].reshape(NCORES, NSUB, ngrp, GRP, CHUNK)
    a_src2 = a_src.reshape(1, d)
    a_dst2 = a_dst.reshape(1, d)

    p = dnm = None
    for layer in range(NUM_LAYERS):
        if layer == 0:
            h, as3, ad3, g = prep_x(features, W, a_src2, a_dst2)
        else:
            h, as3, ad3, g = prep_p(p, p, dnm, dnm, W, a_src2, a_dst2)
        pflat, dflat = sc_edge(h, as3.reshape(n), ad3.reshape(n), g[0:1], src4, dst4)
        p = pflat
        dnm = dflat.reshape(NCORES, np_pad, 1)
    return combine(p, p, dnm, dnm)


# speedup vs baseline: 1.0179x; 1.0179x over previous
import functools

import jax
import jax.numpy as jnp
from jax import lax
from jax.experimental import pallas as pl
from jax.experimental.pallas import tpu as pltpu
from jax.experimental.pallas import tpu_sc as plsc

NCORES = 2
NSUB = 16
LANES = 16
CHUNK = 80
BM = 1000
NUM_LAYERS = 3


def _elu(v):
    return jnp.where(v > 0, v, jnp.exp(v) - 1.0)


def _alphas_and_gmax(i, h, a_src_ref, a_dst_ref, as_ref, ad_ref, g_ref):
    as_blk = jnp.dot(h, a_src_ref[0, :], preferred_element_type=jnp.float32)
    ad_blk = jnp.dot(h, a_dst_ref[0, :], preferred_element_type=jnp.float32)
    as_ref[0, 0, :] = as_blk
    ad_ref[0, 0, :] = ad_blk

    @pl.when(i == 0)
    def _():
        g_ref[...] = jnp.full((8, 128), -jnp.inf, jnp.float32)

    g_ref[...] = jnp.maximum(g_ref[...], jnp.full((8, 128), jnp.max(as_blk)))


def _prep_x_body(x_ref, w_ref, a_src_ref, a_dst_ref, h_ref, as_ref, ad_ref, g_ref):
    i = pl.program_id(0)
    h = jnp.dot(x_ref[...], w_ref[...], preferred_element_type=jnp.float32)
    h_ref[...] = h
    _alphas_and_gmax(i, h, a_src_ref, a_dst_ref, as_ref, ad_ref, g_ref)


def _prep_p_body(p_ref0, p_ref1, d_ref0, d_ref1, w_ref, a_src_ref, a_dst_ref,
                 h_ref, as_ref, ad_ref, g_ref):
    i = pl.program_id(0)
    num = p_ref0[0, :, :] + p_ref1[0, :, :]
    den = d_ref0[0, :, :] + d_ref1[0, :, :] + 1e-16
    x = _elu(num / den)
    h = jnp.dot(x, w_ref[...], preferred_element_type=jnp.float32)
    h_ref[...] = h
    _alphas_and_gmax(i, h, a_src_ref, a_dst_ref, as_ref, ad_ref, g_ref)


def _combine_body(p_ref0, p_ref1, d_ref0, d_ref1, o_ref):
    num = p_ref0[0, :, :] + p_ref1[0, :, :]
    den = d_ref0[0, :, :] + d_ref1[0, :, :] + 1e-16
    o_ref[...] = _elu(num / den)


def _make_tc_kernels(n, d, np_pad):
    nb = n // BM
    w_spec = pl.BlockSpec((d, d), lambda i: (0, 0))
    a_spec = pl.BlockSpec((1, d), lambda i: (0, 0))
    x_spec = pl.BlockSpec((BM, d), lambda i: (i, 0))
    p0_spec = pl.BlockSpec((1, BM, d), lambda i: (0, i, 0))
    p1_spec = pl.BlockSpec((1, BM, d), lambda i: (1, i, 0))
    d0_spec = pl.BlockSpec((1, BM, 1), lambda i: (0, i, 0))
    d1_spec = pl.BlockSpec((1, BM, 1), lambda i: (1, i, 0))
    al_spec = pl.BlockSpec((1, 1, BM), lambda i: (i, 0, 0))
    g_spec = pl.BlockSpec((8, 128), lambda i: (0, 0))

    out_types = [
        jax.ShapeDtypeStruct((n, d), jnp.float32),
        jax.ShapeDtypeStruct((nb, 1, BM), jnp.float32),
        jax.ShapeDtypeStruct((nb, 1, BM), jnp.float32),
        jax.ShapeDtypeStruct((8, 128), jnp.float32),
    ]
    out_specs = [x_spec, al_spec, al_spec, g_spec]

    prep_x = pl.pallas_call(
        _prep_x_body,
        grid=(nb,),
        in_specs=[x_spec, w_spec, a_spec, a_spec],
        out_specs=out_specs,
        out_shape=out_types,
    )
    prep_p = pl.pallas_call(
        _prep_p_body,
        grid=(nb,),
        in_specs=[p0_spec, p1_spec, d0_spec, d1_spec, w_spec, a_spec, a_spec],
        out_specs=out_specs,
        out_shape=out_types,
    )
    combine = pl.pallas_call(
        _combine_body,
        grid=(nb,),
        in_specs=[p0_spec, p1_spec, d0_spec, d1_spec],
        out_specs=x_spec,
        out_shape=jax.ShapeDtypeStruct((n, d), jnp.float32),
    )
    return prep_x, prep_p, combine


GRP = 25


def _make_sc_edge(n, d, e, np_pad):
    per_tile = e // (NCORES * NSUB)
    ngrp = per_tile // (GRP * CHUNK)
    assert ngrp * GRP * CHUNK * NCORES * NSUB == e
    rows_per_tile = np_pad // NSUB
    nzc = rows_per_tile // CHUNK
    assert nzc * CHUNK == rows_per_tile

    mesh = plsc.VectorSubcoreMesh(
        core_axis_name="c", subcore_axis_name="s",
        num_cores=NCORES, num_subcores=NSUB)

    @functools.partial(
        pl.kernel,
        out_type=[
            jax.ShapeDtypeStruct((NCORES, np_pad, d), jnp.float32),
            jax.ShapeDtypeStruct((NCORES, np_pad), jnp.float32),
        ],
        mesh=mesh,
        compiler_params=pltpu.CompilerParams(needs_layout_passes=False),
        scratch_types=[
            pltpu.VMEM((1, 128), jnp.float32),
            pltpu.VMEM((2, GRP, CHUNK), jnp.int32),
            pltpu.VMEM((2, GRP, CHUNK), jnp.int32),
            pltpu.VMEM((2, CHUNK, d), jnp.float32),
            pltpu.VMEM((2, CHUNK), jnp.float32),
            pltpu.VMEM((2, CHUNK), jnp.float32),
            pltpu.VMEM((2, CHUNK), jnp.float32),
            pltpu.VMEM_SHARED((np_pad, d), jnp.float32),
            pltpu.VMEM_SHARED((np_pad,), jnp.float32),
            pltpu.SemaphoreType.DMA((2,)),
            pltpu.SemaphoreType.DMA((2,)),
            pltpu.SemaphoreType.DMA((2,)),
        ],
    )
    def sc_edge(h_hbm, as_hbm, ad_hbm, g_hbm, src_hbm, dst_hbm,
                p_out, den_out, g_v, src_g, dst_g, rows2, asc2, adc2, ee2,
                p_sp, d_sp, isem, gsem, ssem):
        cid = lax.axis_index("c")
        sid = lax.axis_index("s")
        row0 = sid * rows_per_tile
        nch = ngrp * GRP

        def _zrows(i, _):
            for u in range(d // LANES):
                rows2[0, i, pl.ds(u * LANES, LANES)] = jnp.zeros((LANES,), jnp.float32)
            return 0
        lax.fori_loop(0, CHUNK, _zrows, 0)
        for u in range(CHUNK // LANES):
            ee2[0, pl.ds(u * LANES, LANES)] = jnp.zeros((LANES,), jnp.float32)
        for b in range(nzc):
            pltpu.sync_copy(rows2.at[0], p_sp.at[pl.ds(row0 + b * CHUNK, CHUNK)])
            pltpu.sync_copy(ee2.at[0], d_sp.at[pl.ds(row0 + b * CHUNK, CHUNK)])

        pltpu.sync_copy(g_hbm, g_v)
        plsc.subcore_barrier()

        g16 = g_v[0, pl.ds(0, LANES)]

        def issue_idx(gi, parity):
            pltpu.async_copy(src_hbm.at[cid, sid, gi], src_g.at[parity],
                             isem.at[parity])
            pltpu.async_copy(dst_hbm.at[cid, sid, gi], dst_g.at[parity],
                             isem.at[parity])

        def wait_idx(parity):
            pltpu.make_async_copy(src_hbm.at[cid, sid, 0], src_g.at[parity],
                                  isem.at[parity]).wait()
            pltpu.make_async_copy(dst_hbm.at[cid, sid, 0], dst_g.at[parity],
                                  isem.at[parity]).wait()

        def wait_gathers(buf):
            pltpu.make_async_copy(h_hbm.at[src_g.at[0, 0]], rows2.at[buf],
                                  gsem.at[buf]).wait()
            pltpu.make_async_copy(as_hbm.at[src_g.at[0, 0]], asc2.at[buf],
                                  gsem.at[buf]).wait()
            pltpu.make_async_copy(ad_hbm.at[dst_g.at[0, 0]], adc2.at[buf],
                                  gsem.at[buf]).wait()

        def wait_scatters(buf):
            pltpu.make_async_copy(rows2.at[buf], p_sp.at[dst_g.at[0, 0]],
                                  ssem.at[buf]).wait()
            pltpu.make_async_copy(ee2.at[buf], d_sp.at[dst_g.at[0, 0]],
                                  ssem.at[buf]).wait()

        def issue_gathers(jc, buf):
            gi = jc // GRP
            jj = jc - gi * GRP
            gp = lax.rem(gi, 2)
            pltpu.async_copy(h_hbm.at[src_g.at[gp, jj]], rows2.at[buf],
                             gsem.at[buf])
            pltpu.async_copy(as_hbm.at[src_g.at[gp, jj]], asc2.at[buf],
                             gsem.at[buf])
            pltpu.async_copy(ad_hbm.at[dst_g.at[gp, jj]], adc2.at[buf],
                             gsem.at[buf])

        issue_idx(0, 0)
        wait_idx(0)
        issue_gathers(0, 0)

        def body(j, _):
            b = lax.rem(j, 2)
            nb = 1 - b
            gi = j // GRP
            jj = j - gi * GRP

            @pl.when(jnp.logical_and(jj == 0, gi + 1 < ngrp))
            def _():
                issue_idx(gi + 1, lax.rem(gi + 1, 2))

            wait_gathers(b)

            @pl.when(j > 0)
            def _():
                wait_scatters(nb)

            @pl.when(j + 1 < nch)
            def _():
                gi1 = (j + 1) // GRP
                jj1 = (j + 1) - gi1 * GRP

                @pl.when(jnp.logical_and(jj1 == 0, gi1 > 0))
                def _():
                    wait_idx(lax.rem(gi1, 2))
                issue_gathers(j + 1, nb)

            for q in range(CHUNK // LANES):
                sl = pl.ds(q * LANES, LANES)
                a_s = asc2[b, sl]
                a_d = adc2[b, sl]
                s = a_s + a_d
                ee = jnp.exp(jnp.where(s > 0, s, 0.2 * s)
                             - jnp.where(a_d + g16 > 0, a_d + g16,
                                         0.2 * (a_d + g16)))
                ee2[b, sl] = ee

            def scale(q, _):
                ee16 = ee2[b, pl.ds(q * LANES, LANES)]
                base = q * LANES
                for r in range(LANES):
                    cf = jnp.full((LANES,), ee16[r], jnp.float32)
                    for u in range(d // LANES):
                        sl = pl.ds(u * LANES, LANES)
                        rows2[b, base + r, sl] = rows2[b, base + r, sl] * cf
                return 0
            lax.fori_loop(0, CHUNK // LANES, scale, 0)

            gp = lax.rem(gi, 2)
            pltpu.async_copy(rows2.at[b], p_sp.at[dst_g.at[gp, jj]],
                             ssem.at[b], add=True)
            pltpu.async_copy(ee2.at[b], d_sp.at[dst_g.at[gp, jj]],
                             ssem.at[b], add=True)
            return 0
        lax.fori_loop(0, nch, body, 0)

        wait_scatters((nch - 1) % 2)
        plsc.subcore_barrier()

        pltpu.sync_copy(p_sp.at[pl.ds(row0, rows_per_tile)],
                        p_out.at[cid, pl.ds(row0, rows_per_tile)])
        pltpu.sync_copy(d_sp.at[pl.ds(row0, rows_per_tile)],
                        den_out.at[cid, pl.ds(row0, rows_per_tile)])

    return sc_edge


def kernel(features, edge_index, W, a_src, a_dst):
    n, d = features.shape
    e = edge_index.shape[1]
    np_pad = ((n + NSUB * CHUNK - 1) // (NSUB * CHUNK)) * (NSUB * CHUNK)

    prep_x, prep_p, combine = _make_tc_kernels(n, d, np_pad)
    sc_edge = _make_sc_edge(n, d, e, np_pad)

    ngrp = e // (NCORES * NSUB * GRP * CHUNK)
    src4 = edge_index[0].reshape(NCORES, NSUB, ngrp, GRP, CHUNK)
    dst4 = edge_index[1].reshape(NCORES, NSUB, ngrp, GRP, CHUNK)
    a_src2 = a_src.reshape(1, d)
    a_dst2 = a_dst.reshape(1, d)

    p = dnm = None
    for layer in range(NUM_LAYERS):
        if layer == 0:
            h, as3, ad3, g = prep_x(features, W, a_src2, a_dst2)
        else:
            h, as3, ad3, g = prep_p(p, p, dnm, dnm, W, a_src2, a_dst2)
        pflat, dflat = sc_edge(h, as3.reshape(n), ad3.reshape(n), g[0:1], src4, dst4)
        p = pflat
        dnm = dflat.reshape(NCORES, np_pad, 1)
    return combine(p, p, dnm, dnm)


# speedup vs baseline: 1.7538x; 1.7229x over previous
import functools

import jax
import jax.numpy as jnp
from jax import lax
from jax.experimental import pallas as pl
from jax.experimental.pallas import tpu as pltpu
from jax.experimental.pallas import tpu_sc as plsc

NCORES = 2
NSUB = 16
LANES = 16
CHUNK = 80
HALF = 40
GRP = 5
BM = 1000
NUM_LAYERS = 3


def _elu(v):
    return jnp.where(v > 0, v, jnp.exp(v) - 1.0)


def _alphas_and_gmax(i, h, a_src_ref, a_dst_ref, as_ref, ad_ref, g_ref):
    as_blk = jnp.dot(h, a_src_ref[0, :], preferred_element_type=jnp.float32)
    ad_blk = jnp.dot(h, a_dst_ref[0, :], preferred_element_type=jnp.float32)
    as_ref[0, 0, :] = as_blk
    ad_ref[0, 0, :] = ad_blk

    @pl.when(i == 0)
    def _():
        g_ref[...] = jnp.full((8, 128), -jnp.inf, jnp.float32)

    g_ref[...] = jnp.maximum(g_ref[...], jnp.full((8, 128), jnp.max(as_blk)))


def _prep_x_body(x_ref, w_ref, a_src_ref, a_dst_ref, h_ref, as_ref, ad_ref, g_ref):
    i = pl.program_id(0)
    h = jnp.dot(x_ref[...], w_ref[...], preferred_element_type=jnp.float32)
    h_ref[...] = h
    _alphas_and_gmax(i, h, a_src_ref, a_dst_ref, as_ref, ad_ref, g_ref)


def _prep_p_body(p_ref0, p_ref1, d_ref0, d_ref1, w_ref, a_src_ref, a_dst_ref,
                 h_ref, as_ref, ad_ref, g_ref):
    i = pl.program_id(0)
    num = p_ref0[0, :, :] + p_ref1[0, :, :]
    den = d_ref0[0, :, :] + d_ref1[0, :, :] + 1e-16
    x = _elu(num / den)
    h = jnp.dot(x, w_ref[...], preferred_element_type=jnp.float32)
    h_ref[...] = h
    _alphas_and_gmax(i, h, a_src_ref, a_dst_ref, as_ref, ad_ref, g_ref)


def _combine_body(p_ref0, p_ref1, d_ref0, d_ref1, o_ref):
    num = p_ref0[0, :, :] + p_ref1[0, :, :]
    den = d_ref0[0, :, :] + d_ref1[0, :, :] + 1e-16
    o_ref[...] = _elu(num / den)


def _make_tc_kernels(n, d, np_pad):
    nb = n // BM
    w_spec = pl.BlockSpec((d, d), lambda i: (0, 0))
    a_spec = pl.BlockSpec((1, d), lambda i: (0, 0))
    x_spec = pl.BlockSpec((BM, d), lambda i: (i, 0))
    p0_spec = pl.BlockSpec((1, BM, d), lambda i: (0, i, 0))
    p1_spec = pl.BlockSpec((1, BM, d), lambda i: (1, i, 0))
    d0_spec = pl.BlockSpec((1, BM, 1), lambda i: (0, i, 0))
    d1_spec = pl.BlockSpec((1, BM, 1), lambda i: (1, i, 0))
    al_spec = pl.BlockSpec((1, 1, BM), lambda i: (i, 0, 0))
    g_spec = pl.BlockSpec((8, 128), lambda i: (0, 0))

    out_types = [
        jax.ShapeDtypeStruct((n, d), jnp.float32),
        jax.ShapeDtypeStruct((nb, 1, BM), jnp.float32),
        jax.ShapeDtypeStruct((nb, 1, BM), jnp.float32),
        jax.ShapeDtypeStruct((8, 128), jnp.float32),
    ]
    out_specs = [x_spec, al_spec, al_spec, g_spec]

    prep_x = pl.pallas_call(
        _prep_x_body,
        grid=(nb,),
        in_specs=[x_spec, w_spec, a_spec, a_spec],
        out_specs=out_specs,
        out_shape=out_types,
    )
    prep_p = pl.pallas_call(
        _prep_p_body,
        grid=(nb,),
        in_specs=[p0_spec, p1_spec, d0_spec, d1_spec, w_spec, a_spec, a_spec],
        out_specs=out_specs,
        out_shape=out_types,
    )
    combine = pl.pallas_call(
        _combine_body,
        grid=(nb,),
        in_specs=[p0_spec, p1_spec, d0_spec, d1_spec],
        out_specs=x_spec,
        out_shape=jax.ShapeDtypeStruct((n, d), jnp.float32),
    )
    return prep_x, prep_p, combine


def _make_sc_edge(n, d, e, np_pad):
    per_tile = e // (NCORES * NSUB)
    nch = per_tile // CHUNK
    ngrp = nch // GRP
    assert ngrp * GRP * CHUNK * NCORES * NSUB == e
    rows_per_tile = np_pad // NSUB
    nzr = rows_per_tile // HALF
    nzc = rows_per_tile // CHUNK
    assert nzr * HALF == rows_per_tile and nzc * CHUNK == rows_per_tile

    mesh = plsc.VectorSubcoreMesh(
        core_axis_name="c", subcore_axis_name="s",
        num_cores=NCORES, num_subcores=NSUB)

    @functools.partial(
        pl.kernel,
        out_type=[
            jax.ShapeDtypeStruct((NCORES, np_pad, d), jnp.float32),
            jax.ShapeDtypeStruct((NCORES, np_pad), jnp.float32),
        ],
        mesh=mesh,
        compiler_params=pltpu.CompilerParams(needs_layout_passes=False),
        scratch_types=[
            pltpu.VMEM((1, 128), jnp.float32),
            pltpu.VMEM((n,), jnp.float32),
            pltpu.VMEM((n,), jnp.float32),
            pltpu.VMEM((2 * GRP, CHUNK), jnp.int32),
            pltpu.VMEM((2 * GRP, CHUNK), jnp.int32),
            pltpu.VMEM((2, GRP, 2, HALF), jnp.int32),
            pltpu.VMEM((2, HALF, d), jnp.float32),
            pltpu.VMEM((2, CHUNK), jnp.float32),
            pltpu.VMEM_SHARED((np_pad, d), jnp.float32),
            pltpu.VMEM_SHARED((np_pad,), jnp.float32),
            pltpu.SemaphoreType.DMA((2,)),
            pltpu.SemaphoreType.DMA((2,)),
            pltpu.SemaphoreType.DMA((2,)),
            pltpu.SemaphoreType.DMA((2,)),
        ],
    )
    def sc_edge(h_hbm, as_hbm, ad_hbm, g_hbm, src_hbm, dst_hbm, dsth_hbm,
                p_out, den_out, g_v, as_v, ad_v, src_g, dst_g, dst_h,
                rows2, ee2, p_sp, d_sp, isem, gsem, rsem, dsem):
        cid = lax.axis_index("c")
        sid = lax.axis_index("s")
        row0 = sid * rows_per_tile

        def _zrows(i, _):
            for u in range(d // LANES):
                rows2[0, i, pl.ds(u * LANES, LANES)] = jnp.zeros((LANES,), jnp.float32)
            return 0
        lax.fori_loop(0, HALF, _zrows, 0)
        for u in range(CHUNK // LANES):
            ee2[0, pl.ds(u * LANES, LANES)] = jnp.zeros((LANES,), jnp.float32)
        for z in range(nzr):
            pltpu.sync_copy(rows2.at[0], p_sp.at[pl.ds(row0 + z * HALF, HALF)])
        for z in range(nzc):
            pltpu.sync_copy(ee2.at[0], d_sp.at[pl.ds(row0 + z * CHUNK, CHUNK)])

        pltpu.sync_copy(as_hbm, as_v)
        pltpu.sync_copy(ad_hbm, ad_v)
        pltpu.sync_copy(g_hbm, g_v)
        plsc.subcore_barrier()

        g16 = g_v[0, pl.ds(0, LANES)]

        def issue_idx(gi, parity):
            rsl = pl.ds(parity * GRP, GRP)
            pltpu.async_copy(src_hbm.at[cid, sid, gi], src_g.at[rsl],
                             isem.at[parity])
            pltpu.async_copy(dst_hbm.at[cid, sid, gi], dst_g.at[rsl],
                             isem.at[parity])
            pltpu.async_copy(dsth_hbm.at[cid, sid, gi], dst_h.at[parity],
                             isem.at[parity])

        def wait_idx(parity):
            rsl = pl.ds(0, GRP)
            pltpu.make_async_copy(src_hbm.at[cid, sid, 0], src_g.at[rsl],
                                  isem.at[parity]).wait()
            pltpu.make_async_copy(dst_hbm.at[cid, sid, 0], dst_g.at[rsl],
                                  isem.at[parity]).wait()
            pltpu.make_async_copy(dsth_hbm.at[cid, sid, 0], dst_h.at[parity],
                                  isem.at[parity]).wait()

        def issue_rgather(gp, jj, hh, buf):
            pltpu.async_copy(
                h_hbm.at[src_g.at[gp * GRP + jj, pl.ds(hh * HALF, HALF)]],
                rows2.at[buf], gsem.at[buf])

        def wait_rgather(buf):
            pltpu.make_async_copy(
                h_hbm.at[src_g.at[0, pl.ds(0, HALF)]],
                rows2.at[buf], gsem.at[buf]).wait()

        def issue_rscatter(gp, jj, hh, buf):
            pltpu.async_copy(rows2.at[buf], p_sp.at[dst_h.at[gp, jj, hh]],
                             rsem.at[buf], add=True)

        def wait_rscatter(buf):
            pltpu.make_async_copy(rows2.at[buf], p_sp.at[dst_h.at[0, 0, 0]],
                                  rsem.at[buf]).wait()

        def issue_dscatter(bb, gp, jj):
            pltpu.async_copy(ee2.at[bb, pl.ds(0, HALF)],
                             d_sp.at[dst_h.at[gp, jj, 0]], dsem.at[bb], add=True)
            pltpu.async_copy(ee2.at[bb, pl.ds(HALF, HALF)],
                             d_sp.at[dst_h.at[gp, jj, 1]], dsem.at[bb], add=True)

        def wait_dscatter(bb):
            pltpu.make_async_copy(ee2.at[bb, pl.ds(0, HALF)],
                                  d_sp.at[dst_h.at[0, 0, 0]],
                                  dsem.at[bb]).wait()
            pltpu.make_async_copy(ee2.at[bb, pl.ds(HALF, HALF)],
                                  d_sp.at[dst_h.at[0, 0, 1]],
                                  dsem.at[bb]).wait()

        def scale(buf, bb, hh):
            e0 = ee2[bb, pl.ds(hh * HALF, LANES)]
            e1 = ee2[bb, pl.ds(hh * HALF + 16, LANES)]
            e2 = ee2[bb, pl.ds(hh * HALF + 24, LANES)]
            for r in range(LANES):
                cf = jnp.full((LANES,), e0[r], jnp.float32)
                for u in range(d // LANES):
                    sl = pl.ds(u * LANES, LANES)
                    rows2[buf, r, sl] = rows2[buf, r, sl] * cf
            for r in range(LANES):
                cf = jnp.full((LANES,), e1[r], jnp.float32)
                for u in range(d // LANES):
                    sl = pl.ds(u * LANES, LANES)
                    rows2[buf, 16 + r, sl] = rows2[buf, 16 + r, sl] * cf
            for r in range(8):
                cf = jnp.full((LANES,), e2[8 + r], jnp.float32)
                for u in range(d // LANES):
                    sl = pl.ds(u * LANES, LANES)
                    rows2[buf, 32 + r, sl] = rows2[buf, 32 + r, sl] * cf

        issue_idx(0, 0)
        wait_idx(0)
        issue_rgather(0, 0, 0, 0)

        def body(j, _):
            b = lax.rem(j, 2)
            gi = j // GRP
            jj = j - gi * GRP
            gp = lax.rem(gi, 2)

            @pl.when(jnp.logical_and(jj == 0, gi + 1 < ngrp))
            def _():
                issue_idx(gi + 1, lax.rem(gi + 1, 2))

            @pl.when(j >= 2)
            def _():
                wait_dscatter(b)
            row = gp * GRP + jj
            for q in range(CHUNK // LANES):
                sl = pl.ds(q * LANES, LANES)
                s16 = src_g[row, sl]
                d16 = dst_g[row, sl]
                a_s = plsc.load_gather(as_v, [s16])
                a_d = plsc.load_gather(ad_v, [d16])
                s = a_s + a_d
                ee2[b, sl] = jnp.exp(
                    jnp.where(s > 0, s, 0.2 * s)
                    - jnp.where(a_d + g16 > 0, a_d + g16, 0.2 * (a_d + g16)))

            wait_rgather(0)

            @pl.when(j > 0)
            def _():
                wait_rscatter(1)
            issue_rgather(gp, jj, 1, 1)
            scale(0, b, 0)
            issue_rscatter(gp, jj, 0, 0)

            wait_rgather(1)

            @pl.when(j + 1 < nch)
            def _():
                wait_rscatter(0)
                gi1 = (j + 1) // GRP
                jj1 = (j + 1) - gi1 * GRP

                @pl.when(jnp.logical_and(jj1 == 0, gi1 > 0))
                def _():
                    wait_idx(lax.rem(gi1, 2))
                issue_rgather(lax.rem(gi1, 2), jj1, 0, 0)
            scale(1, b, 1)
            issue_rscatter(gp, jj, 1, 1)

            issue_dscatter(b, gp, jj)
            return 0
        lax.fori_loop(0, nch, body, 0)

        wait_rscatter(0)
        wait_rscatter(1)
        wait_dscatter(0)
        wait_dscatter(1)
        plsc.subcore_barrier()

        pltpu.sync_copy(p_sp.at[pl.ds(row0, rows_per_tile)],
                        p_out.at[cid, pl.ds(row0, rows_per_tile)])
        pltpu.sync_copy(d_sp.at[pl.ds(row0, rows_per_tile)],
                        den_out.at[cid, pl.ds(row0, rows_per_tile)])

    return sc_edge


def kernel(features, edge_index, W, a_src, a_dst):
    n, d = features.shape
    e = edge_index.shape[1]
    np_pad = ((n + NSUB * CHUNK - 1) // (NSUB * CHUNK)) * (NSUB * CHUNK)

    prep_x, prep_p, combine = _make_tc_kernels(n, d, np_pad)
    sc_edge = _make_sc_edge(n, d, e, np_pad)

    ngrp = e // (NCORES * NSUB * GRP * CHUNK)
    src5 = edge_index[0].reshape(NCORES, NSUB, ngrp, GRP, CHUNK)
    dst5 = edge_index[1].reshape(NCORES, NSUB, ngrp, GRP, CHUNK)
    dst6 = edge_index[1].reshape(NCORES, NSUB, ngrp, GRP, 2, HALF)
    a_src2 = a_src.reshape(1, d)
    a_dst2 = a_dst.reshape(1, d)

    p = dnm = None
    for layer in range(NUM_LAYERS):
        if layer == 0:
            h, as3, ad3, g = prep_x(features, W, a_src2, a_dst2)
        else:
            h, as3, ad3, g = prep_p(p, p, dnm, dnm, W, a_src2, a_dst2)
        pflat, dflat = sc_edge(h, as3.reshape(n), ad3.reshape(n), g[0:1],
                               src5, dst5, dst6)
        p = pflat
        dnm = dflat.reshape(NCORES, np_pad, 1)
    return combine(p, p, dnm, dnm)


# speedup vs baseline: 2.5169x; 1.4351x over previous
import functools

import jax
import jax.numpy as jnp
from jax import lax
from jax.experimental import pallas as pl
from jax.experimental.pallas import tpu as pltpu
from jax.experimental.pallas import tpu_sc as plsc

NCORES = 2
NSUB = 16
LANES = 16
CHUNK = 80
HALF = 40
GRP = 5
BM = 1000
NUM_LAYERS = 3


def _elu(v):
    return jnp.where(v > 0, v, jnp.exp(v) - 1.0)


def _alphas_and_gmax(i, h, a_src_ref, a_dst_ref, as_ref, ad_ref, g_ref):
    as_blk = jnp.dot(h, a_src_ref[0, :], preferred_element_type=jnp.float32)
    ad_blk = jnp.dot(h, a_dst_ref[0, :], preferred_element_type=jnp.float32)
    as_ref[0, 0, :] = as_blk
    ad_ref[0, 0, :] = ad_blk

    @pl.when(i == 0)
    def _():
        g_ref[...] = jnp.full((8, 128), -jnp.inf, jnp.float32)

    g_ref[...] = jnp.maximum(g_ref[...], jnp.full((8, 128), jnp.max(as_blk)))


def _prep_x_body(x_ref, w_ref, a_src_ref, a_dst_ref, h_ref, as_ref, ad_ref, g_ref):
    i = pl.program_id(0)
    h = jnp.dot(x_ref[...], w_ref[...], preferred_element_type=jnp.float32)
    h_ref[...] = h
    _alphas_and_gmax(i, h, a_src_ref, a_dst_ref, as_ref, ad_ref, g_ref)


def _prep_p_body(p_ref0, p_ref1, d_ref0, d_ref1, w_ref, a_src_ref, a_dst_ref,
                 h_ref, as_ref, ad_ref, g_ref):
    i = pl.program_id(0)
    num = p_ref0[0, :, :] + p_ref1[0, :, :]
    den = d_ref0[0, :, :] + d_ref1[0, :, :] + 1e-16
    x = _elu(num / den)
    h = jnp.dot(x, w_ref[...], preferred_element_type=jnp.float32)
    h_ref[...] = h
    _alphas_and_gmax(i, h, a_src_ref, a_dst_ref, as_ref, ad_ref, g_ref)


def _combine_body(p_ref0, p_ref1, d_ref0, d_ref1, o_ref):
    num = p_ref0[0, :, :] + p_ref1[0, :, :]
    den = d_ref0[0, :, :] + d_ref1[0, :, :] + 1e-16
    o_ref[...] = _elu(num / den)


def _make_tc_kernels(n, d, np_pad):
    nb = n // BM
    w_spec = pl.BlockSpec((d, d), lambda i: (0, 0))
    a_spec = pl.BlockSpec((1, d), lambda i: (0, 0))
    x_spec = pl.BlockSpec((BM, d), lambda i: (i, 0))
    p0_spec = pl.BlockSpec((1, BM, d), lambda i: (0, i, 0))
    p1_spec = pl.BlockSpec((1, BM, d), lambda i: (1, i, 0))
    d0_spec = pl.BlockSpec((1, BM, 1), lambda i: (0, i, 0))
    d1_spec = pl.BlockSpec((1, BM, 1), lambda i: (1, i, 0))
    al_spec = pl.BlockSpec((1, 1, BM), lambda i: (i, 0, 0))
    g_spec = pl.BlockSpec((8, 128), lambda i: (0, 0))

    out_types = [
        jax.ShapeDtypeStruct((n, d), jnp.float32),
        jax.ShapeDtypeStruct((nb, 1, BM), jnp.float32),
        jax.ShapeDtypeStruct((nb, 1, BM), jnp.float32),
        jax.ShapeDtypeStruct((8, 128), jnp.float32),
    ]
    out_specs = [x_spec, al_spec, al_spec, g_spec]

    prep_x = pl.pallas_call(
        _prep_x_body,
        grid=(nb,),
        in_specs=[x_spec, w_spec, a_spec, a_spec],
        out_specs=out_specs,
        out_shape=out_types,
    )
    prep_p = pl.pallas_call(
        _prep_p_body,
        grid=(nb,),
        in_specs=[p0_spec, p1_spec, d0_spec, d1_spec, w_spec, a_spec, a_spec],
        out_specs=out_specs,
        out_shape=out_types,
    )
    combine = pl.pallas_call(
        _combine_body,
        grid=(nb,),
        in_specs=[p0_spec, p1_spec, d0_spec, d1_spec],
        out_specs=x_spec,
        out_shape=jax.ShapeDtypeStruct((n, d), jnp.float32),
    )
    return prep_x, prep_p, combine


def _make_sc_edge(n, d, e, np_pad):
    per_tile = e // (NCORES * NSUB)
    nch = per_tile // CHUNK
    ngrp = nch // GRP
    assert ngrp * GRP * CHUNK * NCORES * NSUB == e
    rows_per_tile = np_pad // NSUB
    nzr = rows_per_tile // HALF
    nzc = rows_per_tile // CHUNK
    assert nzr * HALF == rows_per_tile and nzc * CHUNK == rows_per_tile

    mesh = plsc.VectorSubcoreMesh(
        core_axis_name="c", subcore_axis_name="s",
        num_cores=NCORES, num_subcores=NSUB)

    @functools.partial(
        pl.kernel,
        out_type=[
            jax.ShapeDtypeStruct((NCORES, np_pad, d), jnp.float32),
            jax.ShapeDtypeStruct((NCORES, np_pad), jnp.float32),
        ],
        mesh=mesh,
        compiler_params=pltpu.CompilerParams(needs_layout_passes=False),
        scratch_types=[
            pltpu.VMEM((1, 128), jnp.float32),
            pltpu.VMEM((2 * GRP, CHUNK), jnp.int32),
            pltpu.VMEM((2 * GRP, CHUNK), jnp.int32),
            pltpu.VMEM((2, GRP, 2, HALF), jnp.int32),
            pltpu.VMEM((4, HALF, d), jnp.float32),
            pltpu.VMEM((2, CHUNK), jnp.float32),
            pltpu.VMEM((2, CHUNK), jnp.float32),
            pltpu.VMEM((2, CHUNK), jnp.float32),
            pltpu.VMEM_SHARED((np_pad, d), jnp.float32),
            pltpu.VMEM_SHARED((np_pad,), jnp.float32),
            pltpu.SemaphoreType.DMA((2,)),
            pltpu.SemaphoreType.DMA((4,)),
            pltpu.SemaphoreType.DMA((4,)),
            pltpu.SemaphoreType.DMA((2,)),
            pltpu.SemaphoreType.DMA((2,)),
        ],
    )
    def sc_edge(h_hbm, as_hbm, ad_hbm, g_hbm, src_hbm, dst_hbm, dsth_hbm,
                p_out, den_out, g_v, src_g, dst_g, dst_h,
                rows4, ee2, asc2, adc2, p_sp, d_sp, isem, gsem, rsem, dsem, asem):
        cid = lax.axis_index("c")
        sid = lax.axis_index("s")
        row0 = sid * rows_per_tile

        def _zrows(i, _):
            for u in range(d // LANES):
                rows4[0, i, pl.ds(u * LANES, LANES)] = jnp.zeros((LANES,), jnp.float32)
            return 0
        lax.fori_loop(0, HALF, _zrows, 0)
        for u in range(CHUNK // LANES):
            ee2[0, pl.ds(u * LANES, LANES)] = jnp.zeros((LANES,), jnp.float32)
        for z in range(nzr):
            pltpu.sync_copy(rows4.at[0], p_sp.at[pl.ds(row0 + z * HALF, HALF)])
        for z in range(nzc):
            pltpu.sync_copy(ee2.at[0], d_sp.at[pl.ds(row0 + z * CHUNK, CHUNK)])

        pltpu.sync_copy(g_hbm, g_v)
        plsc.subcore_barrier()

        g16 = g_v[0, pl.ds(0, LANES)]

        def issue_idx(gi, parity):
            rsl = pl.ds(parity * GRP, GRP)
            pltpu.async_copy(src_hbm.at[cid, sid, gi], src_g.at[rsl],
                             isem.at[parity])
            pltpu.async_copy(dst_hbm.at[cid, sid, gi], dst_g.at[rsl],
                             isem.at[parity])
            pltpu.async_copy(dsth_hbm.at[cid, sid, gi], dst_h.at[parity],
                             isem.at[parity])

        def wait_idx(parity):
            rsl = pl.ds(0, GRP)
            pltpu.make_async_copy(src_hbm.at[cid, sid, 0], src_g.at[rsl],
                                  isem.at[parity]).wait()
            pltpu.make_async_copy(dst_hbm.at[cid, sid, 0], dst_g.at[rsl],
                                  isem.at[parity]).wait()
            pltpu.make_async_copy(dsth_hbm.at[cid, sid, 0], dst_h.at[parity],
                                  isem.at[parity]).wait()

        def issue_rgather(gp, jj, hh, buf):
            pltpu.async_copy(
                h_hbm.at[src_g.at[gp * GRP + jj, pl.ds(hh * HALF, HALF)]],
                rows4.at[buf], gsem.at[buf])

        def wait_rgather(buf):
            pltpu.make_async_copy(
                h_hbm.at[src_g.at[0, pl.ds(0, HALF)]],
                rows4.at[buf], gsem.at[buf]).wait()

        def issue_rscatter(gp, jj, hh, buf):
            pltpu.async_copy(rows4.at[buf], p_sp.at[dst_h.at[gp, jj, hh]],
                             rsem.at[buf], add=True)

        def wait_rscatter(buf):
            pltpu.make_async_copy(rows4.at[buf], p_sp.at[dst_h.at[0, 0, 0]],
                                  rsem.at[buf]).wait()

        def issue_alpha(jc, bb):
            gi = jc // GRP
            jj = jc - gi * GRP
            row = lax.rem(gi, 2) * GRP + jj
            pltpu.async_copy(as_hbm.at[src_g.at[row]], asc2.at[bb],
                             asem.at[bb])
            pltpu.async_copy(ad_hbm.at[dst_g.at[row]], adc2.at[bb],
                             asem.at[bb])

        def wait_alpha(bb):
            pltpu.make_async_copy(as_hbm.at[src_g.at[0]], asc2.at[bb],
                                  asem.at[bb]).wait()
            pltpu.make_async_copy(ad_hbm.at[dst_g.at[0]], adc2.at[bb],
                                  asem.at[bb]).wait()

        def issue_dscatter(bb, gp, jj):
            pltpu.async_copy(ee2.at[bb, pl.ds(0, HALF)],
                             d_sp.at[dst_h.at[gp, jj, 0]], dsem.at[bb], add=True)
            pltpu.async_copy(ee2.at[bb, pl.ds(HALF, HALF)],
                             d_sp.at[dst_h.at[gp, jj, 1]], dsem.at[bb], add=True)

        def wait_dscatter(bb):
            pltpu.make_async_copy(ee2.at[bb, pl.ds(0, HALF)],
                                  d_sp.at[dst_h.at[0, 0, 0]],
                                  dsem.at[bb]).wait()
            pltpu.make_async_copy(ee2.at[bb, pl.ds(HALF, HALF)],
                                  d_sp.at[dst_h.at[0, 0, 1]],
                                  dsem.at[bb]).wait()

        def scale(buf, bb, hh):
            e0 = ee2[bb, pl.ds(hh * HALF, LANES)]
            e1 = ee2[bb, pl.ds(hh * HALF + 16, LANES)]
            e2 = ee2[bb, pl.ds(hh * HALF + 24, LANES)]
            for r in range(LANES):
                cf = jnp.full((LANES,), e0[r], jnp.float32)
                for u in range(d // LANES):
                    sl = pl.ds(u * LANES, LANES)
                    rows4[buf, r, sl] = rows4[buf, r, sl] * cf
            for r in range(LANES):
                cf = jnp.full((LANES,), e1[r], jnp.float32)
                for u in range(d // LANES):
                    sl = pl.ds(u * LANES, LANES)
                    rows4[buf, 16 + r, sl] = rows4[buf, 16 + r, sl] * cf
            for r in range(8):
                cf = jnp.full((LANES,), e2[8 + r], jnp.float32)
                for u in range(d // LANES):
                    sl = pl.ds(u * LANES, LANES)
                    rows4[buf, 32 + r, sl] = rows4[buf, 32 + r, sl] * cf

        issue_idx(0, 0)
        wait_idx(0)
        issue_alpha(0, 0)
        issue_rgather(0, 0, 0, 0)
        issue_rgather(0, 0, 1, 1)

        def body(j, _):
            b = lax.rem(j, 2)
            nb = 1 - b
            gi = j // GRP
            jj = j - gi * GRP
            gp = lax.rem(gi, 2)
            buf_a = 2 * b
            buf_b = 2 * b + 1
            nx_a = 2 * nb
            nx_b = 2 * nb + 1

            @pl.when(jnp.logical_and(jj == 2, gi + 1 < ngrp))
            def _():
                issue_idx(gi + 1, lax.rem(gi + 1, 2))

            wait_alpha(b)

            @pl.when(j >= 2)
            def _():
                wait_dscatter(b)
            for q in range(CHUNK // LANES):
                sl = pl.ds(q * LANES, LANES)
                a_s = asc2[b, sl]
                a_d = adc2[b, sl]
                s = a_s + a_d
                ee2[b, sl] = jnp.exp(
                    jnp.where(s > 0, s, 0.2 * s)
                    - jnp.where(a_d + g16 > 0, a_d + g16, 0.2 * (a_d + g16)))

            wait_rgather(buf_a)

            @pl.when(j > 0)
            def _():
                wait_rscatter(nx_a)

            @pl.when(j + 1 < nch)
            def _():
                gi1 = (j + 1) // GRP
                jj1 = (j + 1) - gi1 * GRP

                @pl.when(jnp.logical_and(jj1 == 0, gi1 > 0))
                def _():
                    wait_idx(lax.rem(gi1, 2))
                issue_rgather(lax.rem(gi1, 2), jj1, 0, nx_a)
                issue_alpha(j + 1, nb)
            scale(buf_a, b, 0)
            issue_rscatter(gp, jj, 0, buf_a)

            wait_rgather(buf_b)

            @pl.when(j > 0)
            def _():
                wait_rscatter(nx_b)

            @pl.when(j + 1 < nch)
            def _():
                gi1 = (j + 1) // GRP
                jj1 = (j + 1) - gi1 * GRP
                issue_rgather(lax.rem(gi1, 2), jj1, 1, nx_b)
            scale(buf_b, b, 1)
            issue_rscatter(gp, jj, 1, buf_b)

            issue_dscatter(b, gp, jj)
            return 0
        lax.fori_loop(0, nch, body, 0)

        wait_rscatter((2 * nch - 2) % 4)
        wait_rscatter((2 * nch - 1) % 4)
        wait_dscatter(0)
        wait_dscatter(1)
        plsc.subcore_barrier()

        pltpu.sync_copy(p_sp.at[pl.ds(row0, rows_per_tile)],
                        p_out.at[cid, pl.ds(row0, rows_per_tile)])
        pltpu.sync_copy(d_sp.at[pl.ds(row0, rows_per_tile)],
                        den_out.at[cid, pl.ds(row0, rows_per_tile)])

    return sc_edge


def kernel(features, edge_index, W, a_src, a_dst):
    n, d = features.shape
    e = edge_index.shape[1]
    np_pad = ((n + NSUB * CHUNK - 1) // (NSUB * CHUNK)) * (NSUB * CHUNK)

    prep_x, prep_p, combine = _make_tc_kernels(n, d, np_pad)
    sc_edge = _make_sc_edge(n, d, e, np_pad)

    ngrp = e // (NCORES * NSUB * GRP * CHUNK)
    src5 = edge_index[0].reshape(NCORES, NSUB, ngrp, GRP, CHUNK)
    dst5 = edge_index[1].reshape(NCORES, NSUB, ngrp, GRP, CHUNK)
    dst6 = edge_index[1].reshape(NCORES, NSUB, ngrp, GRP, 2, HALF)
    a_src2 = a_src.reshape(1, d)
    a_dst2 = a_dst.reshape(1, d)

    p = dnm = None
    for layer in range(NUM_LAYERS):
        if layer == 0:
            h, as3, ad3, g = prep_x(features, W, a_src2, a_dst2)
        else:
            h, as3, ad3, g = prep_p(p, p, dnm, dnm, W, a_src2, a_dst2)
        pflat, dflat = sc_edge(h, as3.reshape(n), ad3.reshape(n), g[0:1],
                               src5, dst5, dst6)
        p = pflat
        dnm = dflat.reshape(NCORES, np_pad, 1)
    return combine(p, p, dnm, dnm)
